# baseline (speedup 1.0000x reference)
import functools

import jax
import jax.numpy as jnp
from jax import lax
from jax.experimental import pallas as pl
from jax.experimental.pallas import tpu as pltpu
from jax.experimental.pallas import tpu_sc as plsc

_B = 16384
_DX = 128
_DE = 64
_INFO = plsc.get_sparse_core_info()
_NC = _INFO.num_cores
_NS = _INFO.num_subcores
_NW = _NC * _NS
_CHUNK = _B // _NW
_Q = 128
_NQ = _CHUNK // _Q
_GROUPS = _CHUNK // 16
_TROWS = 56

_mesh = plsc.VectorSubcoreMesh(core_axis_name="c", subcore_axis_name="s")


@functools.partial(
    pl.kernel,
    mesh=_mesh,
    out_type=(
        jax.ShapeDtypeStruct((_B, _DX + _DE), jnp.float32),
        jax.ShapeDtypeStruct((_NW * _TROWS, 2 * _DE), jnp.float32),
    ),
    scratch_types=[
        pltpu.VMEM((4, _CHUNK), jnp.int32),
        pltpu.VMEM((7, _DE), jnp.float32),
        pltpu.VMEM((7, _DE), jnp.float32),
        pltpu.VMEM((7, _DE), jnp.float32),
        pltpu.VMEM((7, _DE), jnp.float32),
        pltpu.VMEM((_TROWS, 2 * _DE), jnp.float32),
        pltpu.VMEM((_NQ, _Q), jnp.int32),
        pltpu.VMEM((_NQ, _Q), jnp.int32),
        pltpu.VMEM((_Q, 2 * _DE), jnp.float32),
        pltpu.VMEM((_Q, 2 * _DE), jnp.float32),
        pltpu.VMEM((2, _Q, _DE), jnp.float32),
        pltpu.VMEM((2, _Q, _DX), jnp.float32),
        pltpu.SemaphoreType.DMA,
        pltpu.SemaphoreType.DMA,
        pltpu.SemaphoreType.DMA,
    ],
)
def _sc_kernel(x_hbm, idx_hbm, t0_hbm, t1_hbm, t2_hbm, t3_hbm,
               out_hbm, d_hbm,
               idx_v, t0_v, t1_v, t2_v, t3_v, d_v,
               a_v, b_v, ra_v, rb_v, emb_v, xq_v, gsem, xsem, osem):
    wid = lax.axis_index("s") * _NC + lax.axis_index("c")
    base = pl.multiple_of(wid * _CHUNK, _CHUNK)
    toff = pl.multiple_of(_TROWS * wid, 8)

    xin = pltpu.async_copy(x_hbm.at[pl.ds(base, _Q), :], xq_v.at[0], xsem)

    pltpu.sync_copy(idx_hbm.at[:, pl.ds(base, _CHUNK)], idx_v)
    pltpu.sync_copy(t0_hbm.at[pl.ds(0, 7), :], t0_v)
    pltpu.sync_copy(t1_hbm.at[pl.ds(0, 7), :], t1_v)
    pltpu.sync_copy(t2_hbm.at[pl.ds(0, 7), :], t2_v)
    pltpu.sync_copy(t3_hbm.at[pl.ds(0, 7), :], t3_v)

    for i in range(7):
        for k in range(_DE // 16):
            a0 = t0_v[i, pl.ds(16 * k, 16)]
            a2 = t2_v[i, pl.ds(16 * k, 16)]
            for j in range(7):
                d_v[7 * i + j, pl.ds(16 * k, 16)] = (
                    a0 + t1_v[j, pl.ds(16 * k, 16)])
                d_v[7 * i + j, pl.ds(_DE + 16 * k, 16)] = (
                    a2 + t3_v[j, pl.ds(16 * k, 16)])

    pltpu.sync_copy(d_v, d_hbm.at[pl.ds(toff, _TROWS), :])

    for g in range(_GROUPS):
        q, m = divmod(g, _GROUPS // _NQ)
        i0 = idx_v[0, pl.ds(16 * g, 16)]
        i1 = idx_v[1, pl.ds(16 * g, 16)]
        i2 = idx_v[2, pl.ds(16 * g, 16)]
        i3 = idx_v[3, pl.ds(16 * g, 16)]
        a_v[q, pl.ds(16 * m, 16)] = toff + 7 * i0 + i1
        b_v[q, pl.ds(16 * m, 16)] = toff + 7 * i2 + i3

    pending = []
    xstore = [None, None]
    estore = [None, None]
    for p in range(_NQ):
        qbase = base + _Q * p
        ga = pltpu.async_copy(d_hbm.at[a_v.at[p]], ra_v, gsem)
        gb = pltpu.async_copy(d_hbm.at[b_v.at[p]], rb_v, gsem)
        xin.wait()
        xs = pltpu.async_copy(
            xq_v.at[p % 2], out_hbm.at[pl.ds(qbase, _Q), pl.ds(0, _DX)], osem)
        pending.append(xs)
        xstore[p % 2] = xs
        if p + 1 < _NQ:
            nxt = (p + 1) % 2
            if xstore[nxt] is not None:
                xstore[nxt].wait()
                pending = [c for c in pending if c is not xstore[nxt]]
                xstore[nxt] = None
            xin = pltpu.async_copy(
                x_hbm.at[pl.ds(qbase + _Q, _Q), :], xq_v.at[nxt], xsem)
        if estore[p % 2] is not None:
            estore[p % 2].wait()
            pending = [c for c in pending if c is not estore[p % 2]]
            estore[p % 2] = None
        ga.wait()
        gb.wait()

        def add_rows(it, carry):
            for u in range(8):
                row = 8 * it + u
                for k in range(_DE // 16):
                    emb_v[p % 2, row, pl.ds(16 * k, 16)] = (
                        ra_v[row, pl.ds(16 * k, 16)]
                        + rb_v[row, pl.ds(_DE + 16 * k, 16)])
            return carry

        lax.fori_loop(0, _Q // 8, add_rows, 0)
        es = pltpu.async_copy(
            emb_v.at[p % 2],
            out_hbm.at[pl.ds(qbase, _Q), pl.ds(_DX, _DE)], osem)
        pending.append(es)
        estore[p % 2] = es

    for s in pending:
        s.wait()


@jax.jit
def kernel(x, x_time_encode, T0, T1, T2, T3):
    idx_t = x_time_encode.T
    out, _ = _sc_kernel(x, idx_t, T0, T1, T2, T3)
    return out


# speedup vs baseline: 1.0698x; 1.0698x over previous
import functools

import jax
import jax.numpy as jnp
from jax import lax
from jax.experimental import pallas as pl
from jax.experimental.pallas import tpu as pltpu
from jax.experimental.pallas import tpu_sc as plsc

_B = 16384
_DX = 128
_DE = 64
_INFO = plsc.get_sparse_core_info()
_NC = _INFO.num_cores
_NS = _INFO.num_subcores
_NW = _NC * _NS
_CHUNK = _B // _NW
_Q = 128
_NQ = _CHUNK // _Q
_GROUPS = _CHUNK // 16
_CROWS = 2432
_BROWS = _CROWS // _NS

_mesh = plsc.VectorSubcoreMesh(core_axis_name="c", subcore_axis_name="s")


@functools.partial(
    pl.kernel,
    mesh=_mesh,
    out_type=(
        jax.ShapeDtypeStruct((_B, _DX + _DE), jnp.float32),
        jax.ShapeDtypeStruct((_NC * _CROWS, 2 * _DE), jnp.float32),
    ),
    scratch_types=[
        pltpu.VMEM((4, _CHUNK), jnp.int32),
        pltpu.VMEM((7, _DE), jnp.float32),
        pltpu.VMEM((7, _DE), jnp.float32),
        pltpu.VMEM((7, _DE), jnp.float32),
        pltpu.VMEM((7, _DE), jnp.float32),
        pltpu.VMEM((49, _DE), jnp.float32),
        pltpu.VMEM((49, _DE), jnp.float32),
        pltpu.VMEM((_BROWS, 2 * _DE), jnp.float32),
        pltpu.VMEM((_NQ, _Q), jnp.int32),
        pltpu.VMEM((_Q, 2 * _DE), jnp.float32),
        pltpu.VMEM((2, _Q, _DE), jnp.float32),
        pltpu.VMEM((2, _Q, _DX), jnp.float32),
        pltpu.SemaphoreType.DMA,
        pltpu.SemaphoreType.DMA,
        pltpu.SemaphoreType.DMA,
    ],
)
def _sc_kernel(x_hbm, idx_hbm, t0_hbm, t1_hbm, t2_hbm, t3_hbm,
               out_hbm, c_hbm,
               idx_v, t0_v, t1_v, t2_v, t3_v, c01_v, c23_v, bld_v,
               cidx_v, r_v, emb_v, xq_v, gsem, xsem, osem):
    cid = lax.axis_index("c")
    sid = lax.axis_index("s")
    wid = sid * _NC + cid
    base = pl.multiple_of(wid * _CHUNK, _CHUNK)

    xin = pltpu.async_copy(x_hbm.at[pl.ds(base, _Q), :], xq_v.at[0], xsem)

    pltpu.sync_copy(idx_hbm.at[:, pl.ds(base, _CHUNK)], idx_v)
    pltpu.sync_copy(t0_hbm.at[pl.ds(0, 7), :], t0_v)
    pltpu.sync_copy(t1_hbm.at[pl.ds(0, 7), :], t1_v)
    pltpu.sync_copy(t2_hbm.at[pl.ds(0, 7), :], t2_v)
    pltpu.sync_copy(t3_hbm.at[pl.ds(0, 7), :], t3_v)

    for i in range(7):
        for k in range(_DE // 16):
            a0 = t0_v[i, pl.ds(16 * k, 16)]
            a2 = t2_v[i, pl.ds(16 * k, 16)]
            for j in range(7):
                c01_v[7 * i + j, pl.ds(16 * k, 16)] = (
                    a0 + t1_v[j, pl.ds(16 * k, 16)])
                c23_v[7 * i + j, pl.ds(16 * k, 16)] = (
                    a2 + t3_v[j, pl.ds(16 * k, 16)])

    def build_row(r, carry):
        a = lax.rem(_BROWS * sid + r, 49)
        b = lax.div(_BROWS * sid + r, 49)
        bm = jnp.where(b > 48, 0, b)
        for k in range(_DE // 16):
            bld_v[r, pl.ds(16 * k, 16)] = (
                c01_v[a, pl.ds(16 * k, 16)] + c23_v[bm, pl.ds(16 * k, 16)])
        return carry

    lax.fori_loop(0, _BROWS, build_row, 0)
    coff = pl.multiple_of(_CROWS * cid + _BROWS * sid, 8)
    pltpu.sync_copy(bld_v, c_hbm.at[pl.ds(coff, _BROWS), :])

    tbase = _CROWS * cid
    for g in range(_GROUPS):
        q, m = divmod(g, _GROUPS // _NQ)
        i0 = idx_v[0, pl.ds(16 * g, 16)]
        i1 = idx_v[1, pl.ds(16 * g, 16)]
        i2 = idx_v[2, pl.ds(16 * g, 16)]
        i3 = idx_v[3, pl.ds(16 * g, 16)]
        cidx_v[q, pl.ds(16 * m, 16)] = (
            tbase + (7 * i0 + i1) + 49 * (7 * i2 + i3))

    plsc.subcore_barrier()

    pending = []
    xstore = [None, None]
    gstore = [None, None]
    for p in range(_NQ):
        qbase = base + _Q * p
        g = pltpu.async_copy(c_hbm.at[cidx_v.at[p]], r_v, gsem)
        xin.wait()
        xs = pltpu.async_copy(
            xq_v.at[p % 2], out_hbm.at[pl.ds(qbase, _Q), pl.ds(0, _DX)], osem)
        pending.append(xs)
        xstore[p % 2] = xs
        if p + 1 < _NQ:
            nxt = (p + 1) % 2
            if xstore[nxt] is not None:
                xstore[nxt].wait()
                pending = [c for c in pending if c is not xstore[nxt]]
                xstore[nxt] = None
            xin = pltpu.async_copy(
                x_hbm.at[pl.ds(qbase + _Q, _Q), :], xq_v.at[nxt], xsem)
        if gstore[p % 2] is not None:
            gstore[p % 2].wait()
            pending = [c for c in pending if c is not gstore[p % 2]]
            gstore[p % 2] = None
        g.wait()

        def copy_rows(it, carry):
            for u in range(8):
                row = 8 * it + u
                for k in range(_DE // 16):
                    emb_v[p % 2, row, pl.ds(16 * k, 16)] = (
                        r_v[row, pl.ds(16 * k, 16)])
            return carry

        lax.fori_loop(0, _Q // 8, copy_rows, 0)
        es = pltpu.async_copy(
            emb_v.at[p % 2],
            out_hbm.at[pl.ds(qbase, _Q), pl.ds(_DX, _DE)], osem)
        pending.append(es)
        gstore[p % 2] = es

    for s in pending:
        s.wait()


@jax.jit
def kernel(x, x_time_encode, T0, T1, T2, T3):
    idx_t = x_time_encode.T
    out, _ = _sc_kernel(x, idx_t, T0, T1, T2, T3)
    return out


# speedup vs baseline: 2.6102x; 2.4399x over previous
import functools

import jax
import jax.numpy as jnp
import numpy as np
from jax.experimental import pallas as pl
from jax.experimental.pallas import tpu as pltpu

_B = 16384
_DX = 128
_DE = 64
_BLK = 2048

_EXPAND = np.equal.outer(np.arange(4), np.arange(32) // 8).astype(np.float32)


def _body(x_ref, idx_ref, s_ref, e_ref, out_ref):
    idxf = idx_ref[...].astype(jnp.float32)
    sel = jnp.dot(idxf, e_ref[...], preferred_element_type=jnp.float32)
    slot = (jax.lax.broadcasted_iota(jnp.int32, (_BLK, 32), 1) % 8
            ).astype(jnp.float32)
    onehot = (sel == slot).astype(jnp.float32)
    emb_t = jax.lax.dot_general(
        s_ref[...], onehot, (((1,), (1,)), ((), ())),
        preferred_element_type=jnp.float32)
    out_ref[_DX:, :] = emb_t
    out_ref[:_DX, :] = x_ref[...].T


def _stacked_table_t(T0, T1, T2, T3):
    s = jnp.zeros((32, _DE), jnp.float32)
    for c, t in enumerate((T0, T1, T2, T3)):
        s = jax.lax.dynamic_update_slice(s, t[:7], (8 * c, 0))
    return s.T


@jax.jit
def kernel(x, x_time_encode, T0, T1, T2, T3):
    s_t = _stacked_table_t(T0, T1, T2, T3)
    grid = _B // _BLK
    out_t = pl.pallas_call(
        _body,
        grid=(grid,),
        in_specs=[
            pl.BlockSpec((_BLK, _DX), lambda i: (i, 0)),
            pl.BlockSpec((_BLK, 4), lambda i: (i, 0)),
            pl.BlockSpec((_DE, 32), lambda i: (0, 0)),
            pl.BlockSpec((4, 32), lambda i: (0, 0)),
        ],
        out_specs=pl.BlockSpec((_DX + _DE, _BLK), lambda i: (0, i)),
        out_shape=jax.ShapeDtypeStruct((_DX + _DE, _B), jnp.float32),
    )(x, x_time_encode, s_t, jnp.asarray(_EXPAND))
    return out_t.T
